# baseline (speedup 1.0000x reference)
import functools

import jax
import jax.numpy as jnp
import numpy as np
from jax import lax
from jax.experimental import pallas as pl
from jax.experimental.pallas import tpu as pltpu
from jax.experimental.pallas import tpu_sc as plsc

N = 16777216
NC, NS, L = 2, 16, 16
NW = NC * NS
PER_W = N // NW
CHUNK = 16384
NCHUNK = PER_W // CHUNK
NITER = NCHUNK // 2
UNROLL = 32
TAB = 128

_SEGS = [(0, 30, 0.1, 0.0, 0.0), (30, 50, 1.0, 3.0, 30.0),
         (50, 70, 2.0, 23.0, 50.0), (70, 75, 3.0, 63.0, 70.0),
         (75, 80, 5.0, 78.0, 75.0), (80, 85, 10.0, 103.0, 80.0),
         (85, 90, 30.0, 153.0, 85.0), (90, 95, 40.0, 303.0, 90.0),
         (95, 128, 50.0, 503.0, 95.0)]
_K_OF_SEG = [(0, 2), (2, 6), (6, 10), (10, 11), (11, 12), (12, 13),
             (13, 14), (14, 15), (15, 16)]
_A_NP = np.zeros(L, np.float32)
_S_NP = np.zeros(L, np.float32)
for (_klo, _khi), (_lo, _hi, _s, _a, _t) in zip(_K_OF_SEG, _SEGS):
    _A_NP[_klo:_khi] = np.float32(_a - _s * _t)
    _S_NP[_klo:_khi] = np.float32(_s)

_mesh = plsc.VectorSubcoreMesh(core_axis_name="c", subcore_axis_name="s")


@functools.partial(
    pl.kernel,
    mesh=_mesh,
    compiler_params=pltpu.CompilerParams(needs_layout_passes=False),
    out_type=jax.ShapeDtypeStruct((N,), jnp.float32),
    scratch_types=[
        pltpu.VMEM((L,), jnp.float32),
        pltpu.VMEM((L,), jnp.float32),
        pltpu.VMEM((CHUNK,), jnp.float32),
        pltpu.VMEM((CHUNK,), jnp.float32),
        pltpu.VMEM((CHUNK,), jnp.float32),
        pltpu.VMEM((CHUNK,), jnp.float32),
        pltpu.SemaphoreType.DMA,
        pltpu.SemaphoreType.DMA,
        pltpu.SemaphoreType.DMA,
        pltpu.SemaphoreType.DMA,
    ],
)
def _fm_sc(x_hbm, ta_hbm, ts_hbm, y_hbm, ta_v, ts_v,
           in0, in1, out0, out1, is0, is1, os0, os1):
    wid = lax.axis_index("s") * NC + lax.axis_index("c")
    base = wid * PER_W

    pltpu.sync_copy(ta_hbm, ta_v)
    pltpu.sync_copy(ts_hbm, ts_v)

    a16 = ta_v[pl.ds(0, L)]
    s16 = ts_v[pl.ds(0, L)]

    def compute(src, dst):
        @plsc.parallel_loop(0, CHUNK, step=L, unroll=UNROLL)
        def _pw(o):
            xv = src[pl.ds(o, L)]
            k = jnp.maximum(xv * 0.2 - 4.0, 0.0).astype(jnp.int32)
            av = jnp.take_along_axis(a16, k, axis=0,
                                     mode="promise_in_bounds")
            sv = jnp.take_along_axis(s16, k, axis=0,
                                     mode="promise_in_bounds")
            dst[pl.ds(o, L)] = av + sv * xv

    pltpu.make_async_copy(x_hbm.at[pl.ds(base, CHUNK)], in0, is0).start()
    pltpu.make_async_copy(x_hbm.at[pl.ds(base + CHUNK, CHUNK)], in1, is1).start()

    def body(it, carry):
        for inb, outb, isem, osem, parity in ((in0, out0, is0, os0, 0),
                                              (in1, out1, is1, os1, 1)):
            off = base + (2 * it + parity) * CHUNK
            pltpu.make_async_copy(x_hbm.at[pl.ds(off, CHUNK)], inb, isem).wait()

            @pl.when(it > 0)
            def _wait_prev_out():
                pltpu.make_async_copy(
                    outb, y_hbm.at[pl.ds(off - 2 * CHUNK, CHUNK)], osem).wait()

            compute(inb, outb)
            pltpu.make_async_copy(outb, y_hbm.at[pl.ds(off, CHUNK)], osem).start()

            @pl.when(it + 1 < NITER)
            def _start_next_in():
                pltpu.make_async_copy(
                    x_hbm.at[pl.ds(off + 2 * CHUNK, CHUNK)], inb, isem).start()
        return carry

    lax.fori_loop(0, NITER, body, 0)

    last = base + (NCHUNK - 2) * CHUNK
    pltpu.make_async_copy(out0, y_hbm.at[pl.ds(last, CHUNK)], os0).wait()
    pltpu.make_async_copy(out1, y_hbm.at[pl.ds(last + CHUNK, CHUNK)], os1).wait()


def kernel(x):
    return _fm_sc(x, jnp.asarray(_A_NP), jnp.asarray(_S_NP))


# speedup vs baseline: 2.0918x; 2.0918x over previous
import functools

import jax
import jax.numpy as jnp
import numpy as np
from jax import lax
from jax.experimental import pallas as pl
from jax.experimental.pallas import tpu as pltpu
from jax.experimental.pallas import tpu_sc as plsc

N = 16777216
NC, NS, L = 2, 16, 16
NW = NC * NS
PER_W = N // NW
CHUNK = 16384
NCHUNK = PER_W // CHUNK
NITER = NCHUNK // 2
UNROLL = 8
TAB = 128

_SEGS = [(0, 30, 0.1, 0.0, 0.0), (30, 50, 1.0, 3.0, 30.0),
         (50, 70, 2.0, 23.0, 50.0), (70, 75, 3.0, 63.0, 70.0),
         (75, 80, 5.0, 78.0, 75.0), (80, 85, 10.0, 103.0, 80.0),
         (85, 90, 30.0, 153.0, 85.0), (90, 95, 40.0, 303.0, 90.0),
         (95, 128, 50.0, 503.0, 95.0)]
_K_OF_SEG = [(0, 2), (2, 6), (6, 10), (10, 11), (11, 12), (12, 13),
             (13, 14), (14, 15), (15, 16)]
_A_NP = np.zeros(L, np.float32)
_S_NP = np.zeros(L, np.float32)
for (_klo, _khi), (_lo, _hi, _s, _a, _t) in zip(_K_OF_SEG, _SEGS):
    _A_NP[_klo:_khi] = np.float32(_a - _s * _t)
    _S_NP[_klo:_khi] = np.float32(_s)

_mesh = plsc.VectorSubcoreMesh(core_axis_name="c", subcore_axis_name="s")


@functools.partial(
    pl.kernel,
    mesh=_mesh,
    compiler_params=pltpu.CompilerParams(needs_layout_passes=False),
    out_type=jax.ShapeDtypeStruct((N,), jnp.float32),
    scratch_types=[
        pltpu.VMEM((L,), jnp.float32),
        pltpu.VMEM((L,), jnp.float32),
        pltpu.VMEM((CHUNK,), jnp.float32),
        pltpu.VMEM((CHUNK,), jnp.float32),
        pltpu.VMEM((CHUNK,), jnp.float32),
        pltpu.VMEM((CHUNK,), jnp.float32),
        pltpu.SemaphoreType.DMA,
        pltpu.SemaphoreType.DMA,
        pltpu.SemaphoreType.DMA,
        pltpu.SemaphoreType.DMA,
    ],
)
def _fm_sc(x_hbm, ta_hbm, ts_hbm, y_hbm, ta_v, ts_v,
           in0, in1, out0, out1, is0, is1, os0, os1):
    wid = lax.axis_index("s") * NC + lax.axis_index("c")
    base = wid * PER_W

    pltpu.sync_copy(ta_hbm, ta_v)
    pltpu.sync_copy(ts_hbm, ts_v)

    a16 = ta_v[pl.ds(0, L)]
    s16 = ts_v[pl.ds(0, L)]

    def compute(src, dst):
        @plsc.parallel_loop(0, CHUNK, step=L, unroll=UNROLL)
        def _pw(o):
            xv = src[pl.ds(o, L)]
            k = jnp.maximum(xv * 0.2 - 4.0, 0.0).astype(jnp.int32)
            av = jnp.take_along_axis(a16, k, axis=0,
                                     mode="promise_in_bounds")
            sv = jnp.take_along_axis(s16, k, axis=0,
                                     mode="promise_in_bounds")
            dst[pl.ds(o, L)] = av + sv * xv

    pltpu.make_async_copy(x_hbm.at[pl.ds(base, CHUNK)], in0, is0).start()
    pltpu.make_async_copy(x_hbm.at[pl.ds(base + CHUNK, CHUNK)], in1, is1).start()

    def body(it, carry):
        for inb, outb, isem, osem, parity in ((in0, out0, is0, os0, 0),
                                              (in1, out1, is1, os1, 1)):
            off = base + (2 * it + parity) * CHUNK
            pltpu.make_async_copy(x_hbm.at[pl.ds(off, CHUNK)], inb, isem).wait()

            @pl.when(it > 0)
            def _wait_prev_out():
                pltpu.make_async_copy(
                    outb, y_hbm.at[pl.ds(off - 2 * CHUNK, CHUNK)], osem).wait()

            compute(inb, outb)
            pltpu.make_async_copy(outb, y_hbm.at[pl.ds(off, CHUNK)], osem).start()

            @pl.when(it + 1 < NITER)
            def _start_next_in():
                pltpu.make_async_copy(
                    x_hbm.at[pl.ds(off + 2 * CHUNK, CHUNK)], inb, isem).start()
        return carry

    lax.fori_loop(0, NITER, body, 0)

    last = base + (NCHUNK - 2) * CHUNK
    pltpu.make_async_copy(out0, y_hbm.at[pl.ds(last, CHUNK)], os0).wait()
    pltpu.make_async_copy(out1, y_hbm.at[pl.ds(last + CHUNK, CHUNK)], os1).wait()


def kernel(x):
    return _fm_sc(x, jnp.asarray(_A_NP), jnp.asarray(_S_NP))
